# baseline (speedup 1.0000x reference)
import jax
import jax.numpy as jnp
from jax import lax
from jax.experimental import pallas as pl
from jax.experimental.pallas import tpu as pltpu
from jax.experimental.pallas import tpu_sc as plsc

B = 4096
D = 15360
NC = 2
NS = 16
NW = NC * NS
BPW = B // NW
CH = 8
G = BPW // CH


def _body(x_hbm, table_hbm, out_hbm, idx_v, rows_v, gsem):
    wid = lax.axis_index("s") * NC + lax.axis_index("c")
    base = wid * BPW
    pltpu.sync_copy(x_hbm.at[pl.ds(base, BPW)], idx_v)

    def chunk(g, carry):
        off = g * CH
        pltpu.async_copy(
            table_hbm.at[idx_v.at[pl.ds(off, CH)]], rows_v, gsem
        ).wait()
        pltpu.sync_copy(rows_v, out_hbm.at[pl.ds(base + off, CH)])
        return carry

    lax.fori_loop(0, G, chunk, 0)


def _gather(x, table):
    f = pl.kernel(
        _body,
        out_type=jax.ShapeDtypeStruct((B, D), jnp.float32),
        mesh=plsc.VectorSubcoreMesh(core_axis_name="c", subcore_axis_name="s"),
        scratch_types=[
            pltpu.VMEM((BPW,), jnp.int32),
            pltpu.VMEM((CH, D), jnp.float32),
            pltpu.SemaphoreType.DMA,
        ],
    )
    return f(x, table)


def kernel(x, table):
    out = _gather(x.astype(jnp.int32), table)
    return out.reshape(B, 128, 30, 2, 2)


# speedup vs baseline: 1.0097x; 1.0097x over previous
import jax
import jax.numpy as jnp
from jax import lax
from jax.experimental import pallas as pl
from jax.experimental.pallas import tpu as pltpu
from jax.experimental.pallas import tpu_sc as plsc

B = 4096
D = 15360
NC = 2
NS = 16
NW = NC * NS
BPW = B // NW
CH = 4
G = BPW // CH


def _body(x_hbm, table_hbm, out_hbm, idx_v, rows_v, gsem, ssem):
    wid = lax.axis_index("s") * NC + lax.axis_index("c")
    base = wid * BPW
    pltpu.sync_copy(x_hbm.at[wid], idx_v)

    def start_gather(g, buf):
        pltpu.async_copy(
            table_hbm.at[idx_v.at[g]], rows_v.at[buf], gsem
        )

    def wait_gather():
        pltpu.make_async_copy(
            table_hbm.at[idx_v.at[0]], rows_v.at[0], gsem
        ).wait()

    def start_scatter(g, buf):
        pltpu.async_copy(
            rows_v.at[buf], out_hbm.at[pl.ds(base + g * CH, CH)], ssem
        )

    def wait_scatter():
        pltpu.make_async_copy(
            rows_v.at[0], out_hbm.at[pl.ds(base, CH)], ssem
        ).wait()

    start_gather(0, 0)

    def step(g, carry):
        wait_gather()
        start_scatter(g, g % 2)
        pl.when(g >= 1)(wait_scatter)
        start_gather(g + 1, (g + 1) % 2)
        return carry

    lax.fori_loop(0, G - 1, step, 0)

    wait_gather()
    start_scatter(G - 1, (G - 1) % 2)
    wait_scatter()
    wait_scatter()


def _gather(x, table):
    f = pl.kernel(
        _body,
        out_type=jax.ShapeDtypeStruct((B, D), jnp.float32),
        mesh=plsc.VectorSubcoreMesh(core_axis_name="c", subcore_axis_name="s"),
        scratch_types=[
            pltpu.VMEM((G, CH), jnp.int32),
            pltpu.VMEM((2, CH, D), jnp.float32),
            pltpu.SemaphoreType.DMA,
            pltpu.SemaphoreType.DMA,
        ],
    )
    return f(x, table)


def kernel(x, table):
    out = _gather(x.astype(jnp.int32).reshape(NW, G, CH), table)
    return out.reshape(B, 128, 30, 2, 2)


# speedup vs baseline: 9.1655x; 9.0775x over previous
import jax
import jax.numpy as jnp
from jax import lax
from jax.experimental import pallas as pl
from jax.experimental.pallas import tpu as pltpu
from jax.experimental.pallas import tpu_sc as plsc

B = 4096
D = 15360
NUM_EMB = 5
NC = 2
NS = 16
NW = NC * NS
BPW = B // NW
W = 8


def _body(x_hbm, table_hbm, out_hbm, idx_v, table_l, sem):
    wid = lax.axis_index("s") * NC + lax.axis_index("c")
    base = wid * BPW
    pltpu.sync_copy(x_hbm.at[wid], idx_v)
    pltpu.sync_copy(table_hbm, table_l)

    def start(r):
        vvec = idx_v[pl.ds((r // 16) * 16, 16)]
        v = vvec[r % 16]
        pltpu.async_copy(table_l.at[v], out_hbm.at[base + r], sem)

    def wait_one():
        pltpu.make_async_copy(table_l.at[0], out_hbm.at[base], sem).wait()

    for r in range(W):
        start(r)
    for r in range(W, BPW):
        wait_one()
        start(r)
    for _ in range(W):
        wait_one()


def _gather(x, table_p):
    f = pl.kernel(
        _body,
        out_type=jax.ShapeDtypeStruct((B, 30, 2, 2, 128), jnp.float32),
        compiler_params=pltpu.CompilerParams(use_tc_tiling_on_sc=True),
        mesh=plsc.VectorSubcoreMesh(core_axis_name="c", subcore_axis_name="s"),
        scratch_types=[
            pltpu.VMEM((BPW,), jnp.int32),
            pltpu.VMEM((NUM_EMB, 30, 2, 2, 128), jnp.float32),
            pltpu.SemaphoreType.DMA,
        ],
    )
    return f(x, table_p)


def kernel(x, table):
    table_p = table.reshape(NUM_EMB, 128, 30, 2, 2).transpose(0, 2, 3, 4, 1)
    out_p = _gather(x.astype(jnp.int32).reshape(NW, BPW), table_p)
    return out_p.transpose(0, 4, 1, 2, 3)


# speedup vs baseline: 9.2218x; 1.0061x over previous
import jax
import jax.numpy as jnp
from jax import lax
from jax.experimental import pallas as pl
from jax.experimental.pallas import tpu as pltpu
from jax.experimental.pallas import tpu_sc as plsc

B = 4096
D = 15360
NUM_EMB = 5
NC = 2
NS = 16
NW = NC * NS
BPW = B // NW
W = 16


def _body(x_hbm, table_hbm, out_hbm, idx_v, table_l, sem):
    wid = lax.axis_index("s") * NC + lax.axis_index("c")
    base = wid * BPW
    pltpu.sync_copy(x_hbm.at[wid], idx_v)
    pltpu.sync_copy(table_hbm, table_l)

    def start(r):
        vvec = idx_v[pl.ds((r // 16) * 16, 16)]
        v = vvec[r % 16]
        pltpu.async_copy(table_l.at[v], out_hbm.at[base + r], sem)

    def wait_one():
        pltpu.make_async_copy(table_l.at[0], out_hbm.at[base], sem).wait()

    for r in range(W):
        start(r)
    for r in range(W, BPW):
        wait_one()
        start(r)
    for _ in range(W):
        wait_one()


def _gather(x, table_p):
    f = pl.kernel(
        _body,
        out_type=jax.ShapeDtypeStruct((B, 30, 2, 2, 128), jnp.float32),
        compiler_params=pltpu.CompilerParams(use_tc_tiling_on_sc=True),
        mesh=plsc.VectorSubcoreMesh(core_axis_name="c", subcore_axis_name="s"),
        scratch_types=[
            pltpu.VMEM((BPW,), jnp.int32),
            pltpu.VMEM((NUM_EMB, 30, 2, 2, 128), jnp.float32),
            pltpu.SemaphoreType.DMA,
        ],
    )
    return f(x, table_p)


def kernel(x, table):
    table_p = table.reshape(NUM_EMB, 128, 30, 2, 2).transpose(0, 2, 3, 4, 1)
    out_p = _gather(x.astype(jnp.int32).reshape(NW, BPW), table_p)
    return out_p.transpose(0, 4, 1, 2, 3)


# speedup vs baseline: 10.1292x; 1.0984x over previous
import jax
import jax.numpy as jnp
from jax import lax
from jax.experimental import pallas as pl
from jax.experimental.pallas import tpu as pltpu
from jax.experimental.pallas import tpu_sc as plsc

B = 4096
D = 15360
NUM_EMB = 5
NC = 2
NS = 16
NW = NC * NS
BPW = B // NW
W = 16


def _body(x_hbm, table_hbm, out_hbm, idx_v, table_l, sem):
    wid = lax.axis_index("s") * NC + lax.axis_index("c")
    base = wid * BPW
    pltpu.sync_copy(x_hbm.at[wid], idx_v)
    pltpu.sync_copy(table_hbm, table_l)

    def start(r):
        vvec = idx_v[pl.ds((r // 16) * 16, 16)]
        v = vvec[r % 16]
        pltpu.async_copy(table_l.at[v], out_hbm.at[base + r], sem)

    def wait_one():
        pltpu.make_async_copy(table_l.at[0], out_hbm.at[base], sem).wait()

    for r in range(W):
        start(r)
    for r in range(W, BPW):
        wait_one()
        start(r)
    for _ in range(W):
        wait_one()


def _gather(x, table_p):
    f = pl.kernel(
        _body,
        out_type=jax.ShapeDtypeStruct((B, 30, 2, 2, 128), jnp.float32),
        compiler_params=pltpu.CompilerParams(use_tc_tiling_on_sc=True),
        mesh=plsc.VectorSubcoreMesh(core_axis_name="c", subcore_axis_name="s"),
        scratch_types=[
            pltpu.VMEM((BPW,), jnp.int32),
            pltpu.VMEM((NUM_EMB, 30, 2, 2, 128), jnp.float32),
            pltpu.SemaphoreType.DMA,
        ],
    )
    return f(x, table_p)


def kernel(x, table):
    table_p = (
        table.reshape(NUM_EMB, 128, 120)
        .transpose(0, 2, 1)
        .reshape(NUM_EMB, 30, 2, 2, 128)
    )
    out_p = _gather(x.astype(jnp.int32).reshape(NW, BPW), table_p)
    return out_p.transpose(0, 4, 1, 2, 3)
